# baseline (speedup 1.0000x reference)
import jax
import jax.numpy as jnp
from jax import lax
from jax.experimental import pallas as pl
from jax.experimental.pallas import tpu as pltpu
from jax.experimental.pallas import tpu_sc as plsc

_DIM = 64
_N = 1000000
_K = 16384
_NC = 2
_NS = 16
_NW = _NC * _NS
_L = 16

_F_TC = 4096
_K_SC = _K - _F_TC
_CHUNK = _K_SC // _NW

_DEPTH = 8


def _body(bank_hbm, idx_hbm, out_hbm, idx_v, *scratch):
    bufs = scratch[:_DEPTH]
    out_v = scratch[_DEPTH]
    sems = scratch[_DEPTH + 1:]
    wid = lax.axis_index("s") * _NC + lax.axis_index("c")
    base = wid * _CHUNK
    pltpu.sync_copy(idx_hbm.at[pl.ds(base, _CHUNK)], idx_v)

    iota = lax.iota(jnp.int32, _L)

    def _start(c, slot):
        pltpu.make_async_copy(
            bank_hbm.at[:, pl.ds(c * 128, 128)], bufs[slot], sems[slot]
        ).start()

    def _finish(c, lane, pos, slot):
        blk = bufs[slot]
        pltpu.make_async_copy(
            bank_hbm.at[:, pl.ds(c * 128, 128)], blk, sems[slot]
        ).wait()
        lv = jnp.full((_L,), lane, jnp.int32)
        for t in range(_DIM // _L):
            rows = iota + (t * _L)
            col = plsc.load_gather(blk, [rows, lv])
            plsc.store_scatter(out_v, [rows * _CHUNK + pos], col)

    @pl.loop(0, _CHUNK, step=_L)
    def _(i):
        v = idx_v[pl.ds(i, _L)]
        cs = [lax.shift_right_logical(v[j], 7) for j in range(_L)]
        lanes = [lax.bitwise_and(v[j], 127) for j in range(_L)]
        ip = jnp.maximum(i - _L, 0)
        vp = idx_v[pl.ds(ip, _L)]
        csp = [lax.shift_right_logical(vp[j], 7) for j in range(_L)]
        lanesp = [lax.bitwise_and(vp[j], 127) for j in range(_L)]

        _LAG = _DEPTH - 1
        for j in range(_L):
            _start(cs[j], j % _DEPTH)
            m = j - _LAG
            if m >= 0:
                _finish(cs[m], lanes[m], i + m, m % _DEPTH)
            else:
                jj = j + _L - _LAG

                @pl.when(i > 0)
                def _():
                    _finish(csp[jj], lanesp[jj], ip + jj, jj % _DEPTH)

    iL = _CHUNK - _L
    vl = idx_v[pl.ds(iL, _L)]
    for jj in range(_L - (_DEPTH - 1), _L):
        c = lax.shift_right_logical(vl[jj], 7)
        lane = lax.bitwise_and(vl[jj], 127)
        _finish(c, lane, iL + jj, jj % _DEPTH)

    @pl.loop(0, _DIM)
    def _(d):
        pltpu.sync_copy(
            out_v.at[pl.ds(d * _CHUNK, _CHUNK)],
            out_hbm.at[d].at[pl.ds(base, _CHUNK)],
        )


def _tc_body(idx_s, bank, out_ref, *scratch):
    bufs = scratch[:16]
    sems = scratch[16:]
    step = pl.program_id(0)
    base_n = _K_SC + step * 128
    lane_iota = lax.broadcasted_iota(jnp.int32, (_DIM, 128), 1)

    def start_group(m):
        for j in range(8):
            slot = (m % 2) * 8 + j
            s = idx_s[base_n + m * 8 + j]
            c = lax.shift_right_logical(s, 7)
            pltpu.make_async_copy(
                bank.at[:, pl.ds(c * 128, 128)], bufs[slot], sems[slot]
            ).start()

    def finish_group(m):
        cols = []
        for j in range(8):
            slot = (m % 2) * 8 + j
            s = idx_s[base_n + m * 8 + j]
            c = lax.shift_right_logical(s, 7)
            lane = lax.bitwise_and(s, 127)
            pltpu.make_async_copy(
                bank.at[:, pl.ds(c * 128, 128)], bufs[slot], sems[slot]
            ).wait()
            blk = bufs[slot][...]
            picked = jnp.where(lane_iota == lane, blk, 0.0)
            cols.append(jnp.sum(picked, axis=1, keepdims=True))
        out_ref[:, pl.ds(m * 8, 8)] = jnp.concatenate(cols, axis=1)

    start_group(0)
    for m in range(16):
        if m < 15:
            start_group(m + 1)
        finish_group(m)


def _tc_gather(membank, n_index):
    return pl.pallas_call(
        _tc_body,
        grid=(_F_TC // 128,),
        in_specs=[
            pl.BlockSpec(memory_space=pltpu.SMEM),
            pl.BlockSpec(memory_space=pl.ANY),
        ],
        out_specs=pl.BlockSpec((_DIM, 128), lambda i: (0, i)),
        out_shape=jax.ShapeDtypeStruct((_DIM, _F_TC), jnp.float32),
        scratch_shapes=(
            [pltpu.VMEM((_DIM, 128), jnp.float32)] * 16
            + [pltpu.SemaphoreType.DMA] * 16
        ),
    )(n_index, membank)


def kernel(membank, n_index):
    mesh = plsc.VectorSubcoreMesh(core_axis_name="c", subcore_axis_name="s")
    gathered = pl.kernel(
        _body,
        out_type=jax.ShapeDtypeStruct((_DIM, _K_SC), jnp.float32),
        mesh=mesh,
        compiler_params=pltpu.CompilerParams(needs_layout_passes=False),
        scratch_types=(
            [pltpu.VMEM((_CHUNK,), jnp.int32)]
            + [pltpu.VMEM((_DIM, 128), jnp.float32)] * _DEPTH
            + [pltpu.VMEM((_DIM * _CHUNK,), jnp.float32)]
            + [pltpu.SemaphoreType.DMA] * _DEPTH
        ),
    )
    sc_out = gathered(membank, n_index)
    tc_out = _tc_gather(membank, n_index)
    return jnp.concatenate([sc_out, tc_out], axis=1)


# speedup vs baseline: 2.1483x; 2.1483x over previous
import jax
import jax.numpy as jnp
from jax import lax
from jax.experimental import pallas as pl
from jax.experimental.pallas import tpu as pltpu
from jax.experimental.pallas import tpu_sc as plsc

_DIM = 64
_N = 1000000
_K = 16384
_NC = 2
_NS = 16
_NW = _NC * _NS
_CHUNK = _K // _NW
_L = 16


_DEPTH = 8


def _body(bank_hbm, idx_hbm, out_hbm, idx_v, *scratch):
    bufs = scratch[:_DEPTH]
    out_v = scratch[_DEPTH]
    sems = scratch[_DEPTH + 1:]
    wid = lax.axis_index("s") * _NC + lax.axis_index("c")
    base = wid * _CHUNK
    pltpu.sync_copy(idx_hbm.at[pl.ds(base, _CHUNK)], idx_v)

    iota = lax.iota(jnp.int32, _L)

    def _start(c, slot):
        pltpu.make_async_copy(
            bank_hbm.at[:, pl.ds(c * 128, 128)], bufs[slot], sems[slot]
        ).start()

    def _finish(c, lane, pos, slot):
        blk = bufs[slot]
        pltpu.make_async_copy(
            bank_hbm.at[:, pl.ds(c * 128, 128)], blk, sems[slot]
        ).wait()
        lv = jnp.full((_L,), lane, jnp.int32)
        for t in range(_DIM // _L):
            rows = iota + (t * _L)
            col = plsc.load_gather(blk, [rows, lv])
            plsc.store_scatter(out_v, [rows * _CHUNK + pos], col)

    @pl.loop(0, _CHUNK, step=_L)
    def _(i):
        v = idx_v[pl.ds(i, _L)]
        cs = [lax.shift_right_logical(v[j], 7) for j in range(_L)]
        lanes = [lax.bitwise_and(v[j], 127) for j in range(_L)]
        ip = jnp.maximum(i - _L, 0)
        vp = idx_v[pl.ds(ip, _L)]
        csp = [lax.shift_right_logical(vp[j], 7) for j in range(_L)]
        lanesp = [lax.bitwise_and(vp[j], 127) for j in range(_L)]

        for j in range(_L):
            m = j - _DEPTH
            if m >= 0:
                _finish(cs[m], lanes[m], i + m, m % _DEPTH)
            else:
                jj = j + _L - _DEPTH

                @pl.when(i > 0)
                def _():
                    _finish(csp[jj], lanesp[jj], ip + jj, jj % _DEPTH)

            _start(cs[j], j % _DEPTH)

    iL = _CHUNK - _L
    vl = idx_v[pl.ds(iL, _L)]
    for jj in range(_L - _DEPTH, _L):
        c = lax.shift_right_logical(vl[jj], 7)
        lane = lax.bitwise_and(vl[jj], 127)
        _finish(c, lane, iL + jj, jj % _DEPTH)

    @pl.loop(0, _DIM)
    def _(d):
        pltpu.sync_copy(
            out_v.at[pl.ds(d * _CHUNK, _CHUNK)],
            out_hbm.at[d].at[pl.ds(base, _CHUNK)],
        )


def kernel(membank, n_index):
    mesh = plsc.VectorSubcoreMesh(core_axis_name="c", subcore_axis_name="s")
    gathered = pl.kernel(
        _body,
        out_type=jax.ShapeDtypeStruct((_DIM, _K), jnp.float32),
        mesh=mesh,
        compiler_params=pltpu.CompilerParams(needs_layout_passes=False),
        scratch_types=(
            [pltpu.VMEM((_CHUNK,), jnp.int32)]
            + [pltpu.VMEM((_DIM, 128), jnp.float32)] * _DEPTH
            + [pltpu.VMEM((_DIM * _CHUNK,), jnp.float32)]
            + [pltpu.SemaphoreType.DMA] * _DEPTH
        ),
    )
    return gathered(membank, n_index)
